# Initial kernel scaffold; baseline (speedup 1.0000x reference)
import functools

import jax
import jax.numpy as jnp
from jax.experimental import pallas as pl
from jax.experimental.pallas import tpu as pltpu

E = 8
TOPK = 2
D = 2048
F = 1408
SF = 5632
T = 2048

BLK = 256
NT = (T * TOPK) // BLK + E
P = NT * BLK

MBLK = 256
NFS = 4
FCH = SF // NFS

NEG = -1e30


def _router_kernel(x_ref, gw_ref, logits_ref, w_ref, e_ref, rank_ref, counts_ref, carry):
    j = pl.program_id(0)

    @pl.when(j == 0)
    def _():
        carry[...] = jnp.zeros_like(carry)

    x = x_ref[...]
    logits = jax.lax.dot(x, gw_ref[...],
                         precision=jax.lax.Precision.HIGHEST,
                         preferred_element_type=jnp.float32)
    logits_ref[...] = logits

    lane = jax.lax.broadcasted_iota(jnp.int32, (MBLK, 128), 1)
    row = jax.lax.broadcasted_iota(jnp.int32, (MBLK, MBLK), 0)
    colk = jax.lax.broadcasted_iota(jnp.int32, (MBLK, MBLK), 1)
    valid = lane < E

    lm = jnp.where(valid, logits, NEG)
    m = jnp.max(lm, axis=1, keepdims=True)
    ex = jnp.where(valid, jnp.exp(lm - m), 0.0)
    p = ex / jnp.sum(ex, axis=1, keepdims=True)

    w0 = jnp.max(p, axis=1, keepdims=True)
    e0 = jnp.min(jnp.where(p >= w0, lane, 128), axis=1, keepdims=True)
    oh0 = (lane == e0).astype(jnp.float32)
    p1 = jnp.where(lane == e0, -1.0, p)
    w1 = jnp.max(p1, axis=1, keepdims=True)
    e1 = jnp.min(jnp.where(p1 >= w1, lane, 128), axis=1, keepdims=True)
    oh1 = (lane == e1).astype(jnp.float32)

    tri = (colk < row).astype(jnp.float32)
    base = carry[...]
    prefix0 = jax.lax.dot(tri, oh0, preferred_element_type=jnp.float32)
    rank0 = jnp.sum((prefix0 + base) * oh0, axis=1, keepdims=True)
    cnt0 = jnp.sum(oh0, axis=0, keepdims=True)
    base1 = base + cnt0
    prefix1 = jax.lax.dot(tri, oh1, preferred_element_type=jnp.float32)
    rank1 = jnp.sum((prefix1 + base1) * oh1, axis=1, keepdims=True)
    cnt1 = jnp.sum(oh1, axis=0, keepdims=True)
    newc = base1 + cnt1
    carry[...] = newc
    counts_ref[...] = newc

    lane0 = lane == 0
    lane1 = lane == 1
    w_ref[...] = jnp.where(lane0, w0, 0.0) + jnp.where(lane1, w1, 0.0)
    e_ref[...] = (jnp.where(lane0, e0, 0) + jnp.where(lane1, e1, 0)).astype(jnp.int32)
    rank_ref[...] = (jnp.where(lane0, rank0, 0.0)
                     + jnp.where(lane1, rank1, 0.0)).astype(jnp.int32)


def _router(x, gw_pad):
    grid = (T // MBLK,)
    return pl.pallas_call(
        _router_kernel,
        grid=grid,
        in_specs=[
            pl.BlockSpec((MBLK, D), lambda j: (j, 0)),
            pl.BlockSpec((D, 128), lambda j: (0, 0)),
        ],
        out_specs=[
            pl.BlockSpec((MBLK, 128), lambda j: (j, 0)),
            pl.BlockSpec((MBLK, 128), lambda j: (j, 0)),
            pl.BlockSpec((MBLK, 128), lambda j: (j, 0)),
            pl.BlockSpec((MBLK, 128), lambda j: (j, 0)),
            pl.BlockSpec((1, 128), lambda j: (0, 0)),
        ],
        out_shape=[
            jax.ShapeDtypeStruct((T, 128), jnp.float32),
            jax.ShapeDtypeStruct((T, 128), jnp.float32),
            jax.ShapeDtypeStruct((T, 128), jnp.int32),
            jax.ShapeDtypeStruct((T, 128), jnp.int32),
            jax.ShapeDtypeStruct((1, 128), jnp.float32),
        ],
        scratch_shapes=[pltpu.VMEM((1, 128), jnp.float32)],
    )(x, gw_pad)


def _expert_ffn_kernel(te_ref, xs_ref, wg_ref, wu_ref, wd_ref, ys_ref):
    j = pl.program_id(0)

    @pl.when(j < te_ref[NT])
    def _():
        xb = xs_ref[...].astype(jnp.bfloat16)
        g = jax.lax.dot(xb, wg_ref[0], preferred_element_type=jnp.float32)
        u = jax.lax.dot(xb, wu_ref[0], preferred_element_type=jnp.float32)
        h = (g * jax.nn.sigmoid(g) * u).astype(jnp.bfloat16)
        ys_ref[...] = jax.lax.dot(h, wd_ref[0], preferred_element_type=jnp.float32)


def _expert_ffn(te, xs, wg, wu, wd):
    grid_spec = pltpu.PrefetchScalarGridSpec(
        num_scalar_prefetch=1,
        grid=(NT,),
        in_specs=[
            pl.BlockSpec((BLK, D), lambda j, te: (j, 0)),
            pl.BlockSpec((1, D, F), lambda j, te: (te[j], 0, 0)),
            pl.BlockSpec((1, D, F), lambda j, te: (te[j], 0, 0)),
            pl.BlockSpec((1, F, D), lambda j, te: (te[j], 0, 0)),
        ],
        out_specs=pl.BlockSpec((BLK, D), lambda j, te: (j, 0)),
    )
    return pl.pallas_call(
        _expert_ffn_kernel,
        grid_spec=grid_spec,
        out_shape=jax.ShapeDtypeStruct((P, D), jnp.float32),
    )(te, xs, wg, wu, wd)


def _shared_kernel(x_ref, swg_ref, swu_ref, swd_ref, sg_ref, c0_ref, c1_ref,
                   w_ref, out_ref, acc):
    f = pl.program_id(1)

    @pl.when(f == 0)
    def _():
        acc[...] = jnp.zeros_like(acc)

    xb = x_ref[...].astype(jnp.bfloat16)
    g = jax.lax.dot(xb, swg_ref[...], preferred_element_type=jnp.float32)
    u = jax.lax.dot(xb, swu_ref[...], preferred_element_type=jnp.float32)
    h = (g * jax.nn.sigmoid(g) * u).astype(jnp.bfloat16)
    acc[...] += jax.lax.dot(h, swd_ref[...], preferred_element_type=jnp.float32)

    @pl.when(f == NFS - 1)
    def _():
        gl = jax.lax.dot(xb, sg_ref[...], preferred_element_type=jnp.float32)
        gate = jax.nn.sigmoid(gl[:, 0:1])
        w0 = w_ref[...][:, 0:1]
        w1 = w_ref[...][:, 1:2]
        out_ref[...] = (acc[...] * gate
                        + c0_ref[...] * w0 + c1_ref[...] * w1)


def _shared_combine(x, swg, swu, swd, sg_pad, c0, c1, w_pad):
    grid = (T // MBLK, NFS)
    return pl.pallas_call(
        _shared_kernel,
        grid=grid,
        in_specs=[
            pl.BlockSpec((MBLK, D), lambda m, f: (m, 0)),
            pl.BlockSpec((D, FCH), lambda m, f: (0, f)),
            pl.BlockSpec((D, FCH), lambda m, f: (0, f)),
            pl.BlockSpec((FCH, D), lambda m, f: (f, 0)),
            pl.BlockSpec((D, 128), lambda m, f: (0, 0)),
            pl.BlockSpec((MBLK, D), lambda m, f: (m, 0)),
            pl.BlockSpec((MBLK, D), lambda m, f: (m, 0)),
            pl.BlockSpec((MBLK, 128), lambda m, f: (m, 0)),
        ],
        out_specs=pl.BlockSpec((MBLK, D), lambda m, f: (m, 0)),
        out_shape=jax.ShapeDtypeStruct((T, D), jnp.float32),
        scratch_shapes=[pltpu.VMEM((MBLK, D), jnp.float32)],
    )(x, swg, swu, swd, sg_pad, c0, c1, w_pad)


def kernel(hidden_states, gate_W, expert_Wg, expert_Wu, expert_Wd,
           shared_Wg, shared_Wu, shared_Wd, shared_gate_W):
    x = hidden_states.reshape(T, D)
    gw_pad = jnp.zeros((D, 128), jnp.float32).at[:, :E].set(gate_W)
    sg_pad = jnp.zeros((D, 128), jnp.bfloat16).at[:, :1].set(
        shared_gate_W.astype(jnp.bfloat16))

    logits_pad, w_pad, e_pad, rank_pad, counts_pad = _router(x, gw_pad)

    counts = counts_pad[0, :E].astype(jnp.int32)
    tiles_per = (counts + (BLK - 1)) // BLK
    tstart = jnp.concatenate([jnp.zeros((1,), jnp.int32), jnp.cumsum(tiles_per)])
    row_start = tstart[:E] * BLK
    e0 = e_pad[:, 0]
    e1 = e_pad[:, 1]
    pos0 = row_start[e0] + rank_pad[:, 0]
    pos1 = row_start[e1] + rank_pad[:, 1]
    pos = jnp.concatenate([pos0, pos1])
    token_flat = jnp.concatenate([jnp.arange(T, dtype=jnp.int32)] * 2)
    sorted_token = jnp.zeros((P,), jnp.int32).at[pos].set(token_flat)
    xs = x[sorted_token]

    used = tstart[E]
    te = jnp.minimum(
        (jnp.arange(NT, dtype=jnp.int32)[:, None] >= tstart[1:E + 1][None, :])
        .sum(axis=1).astype(jnp.int32), E - 1)
    te = jnp.concatenate([te, used[None]])

    wg_bf = expert_Wg.astype(jnp.bfloat16)
    wu_bf = expert_Wu.astype(jnp.bfloat16)
    wd_bf = expert_Wd.astype(jnp.bfloat16)
    ys = _expert_ffn(te, xs, wg_bf, wu_bf, wd_bf)

    c0 = ys[pos0]
    c1 = ys[pos1]

    final = _shared_combine(x, shared_Wg.astype(jnp.bfloat16),
                            shared_Wu.astype(jnp.bfloat16),
                            shared_Wd.astype(jnp.bfloat16),
                            sg_pad, c0, c1, w_pad)

    return (final.reshape(1, T, D), logits_pad[:, :E])


# speedup vs baseline: 1.2047x; 1.2047x over previous
import functools

import jax
import jax.numpy as jnp
from jax.experimental import pallas as pl
from jax.experimental.pallas import tpu as pltpu

E = 8
TOPK = 2
D = 2048
F = 1408
SF = 5632
T = 2048

BLK = 256
NT = (T * TOPK) // BLK + E
P = NT * BLK

MBLK = 256
NFS = 4
FCH = SF // NFS

NEG = -1e30


def _router_kernel(x_ref, gw_ref, logits_ref, w_ref, e_ref, rank_ref, counts_ref, carry):
    j = pl.program_id(0)

    @pl.when(j == 0)
    def _():
        carry[...] = jnp.zeros_like(carry)

    x = x_ref[...]
    logits = jax.lax.dot(x, gw_ref[...],
                         preferred_element_type=jnp.float32)
    logits_ref[...] = logits

    lane = jax.lax.broadcasted_iota(jnp.int32, (MBLK, 128), 1)
    row = jax.lax.broadcasted_iota(jnp.int32, (MBLK, MBLK), 0)
    colk = jax.lax.broadcasted_iota(jnp.int32, (MBLK, MBLK), 1)
    valid = lane < E

    lm = jnp.where(valid, logits, NEG)
    m = jnp.max(lm, axis=1, keepdims=True)
    ex = jnp.where(valid, jnp.exp(lm - m), 0.0)
    p = ex / jnp.sum(ex, axis=1, keepdims=True)

    w0 = jnp.max(p, axis=1, keepdims=True)
    e0 = jnp.min(jnp.where(p >= w0, lane, 128), axis=1, keepdims=True)
    oh0 = (lane == e0).astype(jnp.float32)
    p1 = jnp.where(lane == e0, -1.0, p)
    w1 = jnp.max(p1, axis=1, keepdims=True)
    e1 = jnp.min(jnp.where(p1 >= w1, lane, 128), axis=1, keepdims=True)
    oh1 = (lane == e1).astype(jnp.float32)

    tri = (colk < row).astype(jnp.float32)
    base = carry[...]
    prefix0 = jax.lax.dot(tri, oh0, preferred_element_type=jnp.float32)
    rank0 = jnp.sum((prefix0 + base) * oh0, axis=1, keepdims=True)
    cnt0 = jnp.sum(oh0, axis=0, keepdims=True)
    base1 = base + cnt0
    prefix1 = jax.lax.dot(tri, oh1, preferred_element_type=jnp.float32)
    rank1 = jnp.sum((prefix1 + base1) * oh1, axis=1, keepdims=True)
    cnt1 = jnp.sum(oh1, axis=0, keepdims=True)
    newc = base1 + cnt1
    carry[...] = newc
    counts_ref[...] = newc

    lane0 = lane == 0
    lane1 = lane == 1
    w_ref[...] = jnp.where(lane0, w0, 0.0) + jnp.where(lane1, w1, 0.0)
    e_ref[...] = (jnp.where(lane0, e0, 0) + jnp.where(lane1, e1, 0)).astype(jnp.int32)
    rank_ref[...] = (jnp.where(lane0, rank0, 0.0)
                     + jnp.where(lane1, rank1, 0.0)).astype(jnp.int32)


def _router(x, gw_pad):
    grid = (T // MBLK,)
    return pl.pallas_call(
        _router_kernel,
        grid=grid,
        in_specs=[
            pl.BlockSpec((MBLK, D), lambda j: (j, 0)),
            pl.BlockSpec((D, 128), lambda j: (0, 0)),
        ],
        out_specs=[
            pl.BlockSpec((MBLK, 128), lambda j: (j, 0)),
            pl.BlockSpec((MBLK, 128), lambda j: (j, 0)),
            pl.BlockSpec((MBLK, 128), lambda j: (j, 0)),
            pl.BlockSpec((MBLK, 128), lambda j: (j, 0)),
            pl.BlockSpec((1, 128), lambda j: (0, 0)),
        ],
        out_shape=[
            jax.ShapeDtypeStruct((T, 128), jnp.float32),
            jax.ShapeDtypeStruct((T, 128), jnp.float32),
            jax.ShapeDtypeStruct((T, 128), jnp.int32),
            jax.ShapeDtypeStruct((T, 128), jnp.int32),
            jax.ShapeDtypeStruct((1, 128), jnp.float32),
        ],
        scratch_shapes=[pltpu.VMEM((1, 128), jnp.float32)],
    )(x, gw_pad)


def _expert_ffn_kernel(te_ref, xs_ref, wg_ref, wu_ref, wd_ref, ys_ref):
    j = pl.program_id(0)

    @pl.when(j < te_ref[NT])
    def _():
        xb = xs_ref[...].astype(jnp.bfloat16)
        g = jax.lax.dot(xb, wg_ref[0], preferred_element_type=jnp.float32)
        u = jax.lax.dot(xb, wu_ref[0], preferred_element_type=jnp.float32)
        h = (g * jax.nn.sigmoid(g) * u).astype(jnp.bfloat16)
        ys_ref[...] = jax.lax.dot(h, wd_ref[0], preferred_element_type=jnp.float32)


def _expert_ffn(te, xs, wg, wu, wd):
    grid_spec = pltpu.PrefetchScalarGridSpec(
        num_scalar_prefetch=1,
        grid=(NT,),
        in_specs=[
            pl.BlockSpec((BLK, D), lambda j, te: (j, 0)),
            pl.BlockSpec((1, D, F), lambda j, te: (te[j], 0, 0)),
            pl.BlockSpec((1, D, F), lambda j, te: (te[j], 0, 0)),
            pl.BlockSpec((1, F, D), lambda j, te: (te[j], 0, 0)),
        ],
        out_specs=pl.BlockSpec((BLK, D), lambda j, te: (j, 0)),
    )
    return pl.pallas_call(
        _expert_ffn_kernel,
        grid_spec=grid_spec,
        out_shape=jax.ShapeDtypeStruct((P, D), jnp.float32),
    )(te, xs, wg, wu, wd)


def _shared_kernel(x_ref, swg_ref, swu_ref, swd_ref, sg_ref, c0_ref, c1_ref,
                   w_ref, out_ref, acc):
    f = pl.program_id(1)

    @pl.when(f == 0)
    def _():
        acc[...] = jnp.zeros_like(acc)

    xb = x_ref[...].astype(jnp.bfloat16)
    g = jax.lax.dot(xb, swg_ref[...], preferred_element_type=jnp.float32)
    u = jax.lax.dot(xb, swu_ref[...], preferred_element_type=jnp.float32)
    h = (g * jax.nn.sigmoid(g) * u).astype(jnp.bfloat16)
    acc[...] += jax.lax.dot(h, swd_ref[...], preferred_element_type=jnp.float32)

    @pl.when(f == NFS - 1)
    def _():
        gl = jax.lax.dot(xb, sg_ref[...], preferred_element_type=jnp.float32)
        gate = jax.nn.sigmoid(gl[:, 0:1])
        w0 = w_ref[...][:, 0:1]
        w1 = w_ref[...][:, 1:2]
        out_ref[...] = (acc[...] * gate
                        + c0_ref[...] * w0 + c1_ref[...] * w1)


def _shared_combine(x, swg, swu, swd, sg_pad, c0, c1, w_pad):
    grid = (T // MBLK, NFS)
    return pl.pallas_call(
        _shared_kernel,
        grid=grid,
        in_specs=[
            pl.BlockSpec((MBLK, D), lambda m, f: (m, 0)),
            pl.BlockSpec((D, FCH), lambda m, f: (0, f)),
            pl.BlockSpec((D, FCH), lambda m, f: (0, f)),
            pl.BlockSpec((FCH, D), lambda m, f: (f, 0)),
            pl.BlockSpec((D, 128), lambda m, f: (0, 0)),
            pl.BlockSpec((MBLK, D), lambda m, f: (m, 0)),
            pl.BlockSpec((MBLK, D), lambda m, f: (m, 0)),
            pl.BlockSpec((MBLK, 128), lambda m, f: (m, 0)),
        ],
        out_specs=pl.BlockSpec((MBLK, D), lambda m, f: (m, 0)),
        out_shape=jax.ShapeDtypeStruct((T, D), jnp.float32),
        scratch_shapes=[pltpu.VMEM((MBLK, D), jnp.float32)],
    )(x, swg, swu, swd, sg_pad, c0, c1, w_pad)


def kernel(hidden_states, gate_W, expert_Wg, expert_Wu, expert_Wd,
           shared_Wg, shared_Wu, shared_Wd, shared_gate_W):
    x = hidden_states.reshape(T, D)
    gw_pad = jnp.zeros((D, 128), jnp.float32).at[:, :E].set(gate_W)
    sg_pad = jnp.zeros((D, 128), jnp.bfloat16).at[:, :1].set(
        shared_gate_W.astype(jnp.bfloat16))

    logits_pad, w_pad, e_pad, rank_pad, counts_pad = _router(x, gw_pad)

    counts = counts_pad[0, :E].astype(jnp.int32)
    tiles_per = (counts + (BLK - 1)) // BLK
    tstart = jnp.concatenate([jnp.zeros((1,), jnp.int32), jnp.cumsum(tiles_per)])
    row_start = tstart[:E] * BLK
    e0 = e_pad[:, 0]
    e1 = e_pad[:, 1]
    pos0 = row_start[e0] + rank_pad[:, 0]
    pos1 = row_start[e1] + rank_pad[:, 1]
    pos = jnp.concatenate([pos0, pos1])
    token_flat = jnp.concatenate([jnp.arange(T, dtype=jnp.int32)] * 2)
    sorted_token = jnp.zeros((P,), jnp.int32).at[pos].set(token_flat)
    xs = x[sorted_token]

    used = tstart[E]
    te = jnp.minimum(
        (jnp.arange(NT, dtype=jnp.int32)[:, None] >= tstart[1:E + 1][None, :])
        .sum(axis=1).astype(jnp.int32), E - 1)
    te = jnp.concatenate([te, used[None]])

    wg_bf = expert_Wg.astype(jnp.bfloat16)
    wu_bf = expert_Wu.astype(jnp.bfloat16)
    wd_bf = expert_Wd.astype(jnp.bfloat16)
    ys = _expert_ffn(te, xs, wg_bf, wu_bf, wd_bf)

    c0 = ys[pos0]
    c1 = ys[pos1]

    final = _shared_combine(x, shared_Wg.astype(jnp.bfloat16),
                            shared_Wu.astype(jnp.bfloat16),
                            shared_Wd.astype(jnp.bfloat16),
                            sg_pad, c0, c1, w_pad)

    return (final.reshape(1, T, D), logits_pad[:, :E])
